# baseline (speedup 1.0000x reference)
import functools

import jax
import jax.numpy as jnp
from jax import lax
from jax.experimental import pallas as pl
from jax.experimental.pallas import tpu as pltpu
from jax.experimental.pallas import tpu_sc as plsc

BATCH = 8
X_STEPS = 512
Y_STEPS = 4096
D_MODEL = 256

NC = 2
NS = 16
NW = NC * NS
LANES = 16

BLK = 128
BPB = Y_STEPS // BLK
NSLOT = BATCH
VPB = BLK // LANES
NBUF = 3
PRIME = NBUF - 1
ROT = NW // BATCH
ZROWS = 64


def _zero_rows(buf, lo, hi):
    zv = jnp.zeros((LANES,), jnp.float32)

    def body(r, carry):
        for c in range(D_MODEL // LANES):
            buf[r, pl.ds(c * LANES, LANES)] = zv
        return carry

    lax.fori_loop(lo, hi, body, 0)


def _sc_body(table_hbm, idx_hbm, len_hbm, out_hbm, *scratch):
    it = iter(scratch)
    idx_v = next(it)
    gidx = tuple(next(it) for _ in range(NSLOT))
    len_v = next(it)
    bufs = tuple(next(it) for _ in range(NBUF))
    zbuf = next(it)
    gsems = tuple(next(it) for _ in range(NBUF))
    wsems = tuple(next(it) for _ in range(NBUF))
    zsem = next(it)
    isem = next(it)

    w = lax.axis_index("s") * NC + lax.axis_index("c")

    pltpu.sync_copy(len_hbm, len_v)
    lenvec = len_v[...]

    len_s = [lenvec[k] for k in range(BATCH)]
    bat, pos, valid, rem, out_off = [], [], [], [], []
    for j in range(NSLOT):
        bj = jnp.mod(w + j, BATCH)
        lj = len_s[0]
        for k in range(1, BATCH):
            lj = jnp.where(bj == k, len_s[k], lj)
        p = jnp.mod(w - ROT * bj, BPB)
        r = jnp.clip(lj - p * BLK, 0, BLK)
        bat.append(bj)
        pos.append(p)
        rem.append(r)
        valid.append(r > 0)
        out_off.append(bj * Y_STEPS + p * BLK)
        pltpu.async_copy(idx_hbm.at[bj, pl.ds(p * BLK, BLK)],
                         idx_v.at[j], isem)
    for j in range(NSLOT):
        pltpu.make_async_copy(idx_hbm.at[bat[j], pl.ds(pos[j] * BLK, BLK)],
                              idx_v.at[j], isem).wait()

    for j in range(NSLOT):
        roff = lax.broadcast(bat[j] * X_STEPS, (LANES,))
        for v in range(VPB):
            g = idx_v[j, pl.ds(v * LANES, LANES)] + roff
            gidx[j][pl.ds(v * LANES, LANES)] = g

    _zero_rows(zbuf, 0, ZROWS)

    for j in range(NSLOT):
        @pl.when(jnp.logical_not(valid[j]))
        def _(j=j):
            for h in range(BLK // ZROWS):
                pltpu.async_copy(
                    zbuf, out_hbm.at[pl.ds(out_off[j] + h * ZROWS, ZROWS)],
                    zsem)

    def gather(j):
        pltpu.async_copy(table_hbm.at[gidx[j]], bufs[j % NBUF],
                         gsems[j % NBUF])

    def out_at(j):
        return out_hbm.at[pl.ds(out_off[j], BLK)]

    for j in range(min(PRIME, NSLOT)):
        @pl.when(valid[j])
        def _(j=j):
            gather(j)

    for j in range(NSLOT):
        @pl.when(valid[j])
        def _(j=j):
            pltpu.make_async_copy(table_hbm.at[gidx[j]], bufs[j % NBUF],
                                  gsems[j % NBUF]).wait()

        @pl.when(valid[j] & (rem[j] < BLK))
        def _(j=j):
            _zero_rows(bufs[j % NBUF], rem[j], BLK)

        @pl.when(valid[j])
        def _(j=j):
            pltpu.async_copy(bufs[j % NBUF], out_at(j), wsems[j % NBUF])

        if j + PRIME < NSLOT:
            k = j + PRIME - NBUF
            if k >= 0:
                @pl.when(valid[j + PRIME] & valid[k])
                def _(j=j, k=k):
                    pltpu.make_async_copy(bufs[k % NBUF], out_at(k),
                                          wsems[k % NBUF]).wait()

            @pl.when(valid[j + PRIME])
            def _(j=j):
                gather(j + PRIME)

    for k in range(NSLOT):
        cond = valid[k]
        if k + NBUF < NSLOT:
            cond = cond & jnp.logical_not(valid[k + NBUF])

        @pl.when(cond)
        def _(k=k):
            pltpu.make_async_copy(bufs[k % NBUF], out_at(k),
                                  wsems[k % NBUF]).wait()

    for j in range(NSLOT):
        @pl.when(jnp.logical_not(valid[j]))
        def _(j=j):
            for h in range(BLK // ZROWS):
                pltpu.make_async_copy(
                    zbuf, out_hbm.at[pl.ds(out_off[j] + h * ZROWS, ZROWS)],
                    zsem).wait()


@functools.cache
def _sc_call():
    mesh = plsc.VectorSubcoreMesh(
        core_axis_name="c", subcore_axis_name="s",
        num_cores=NC, num_subcores=NS)
    return pl.kernel(
        _sc_body,
        out_type=jax.ShapeDtypeStruct((BATCH * Y_STEPS, D_MODEL), jnp.float32),
        mesh=mesh,
        scratch_types=[
            pltpu.VMEM((NSLOT, BLK), jnp.int32),
            *[pltpu.VMEM((BLK,), jnp.int32) for _ in range(NSLOT)],
            pltpu.VMEM((LANES,), jnp.int32),
            *[pltpu.VMEM((BLK, D_MODEL), jnp.float32) for _ in range(NBUF)],
            pltpu.VMEM((ZROWS, D_MODEL), jnp.float32),
            *[pltpu.SemaphoreType.DMA for _ in range(2 * NBUF + 2)],
        ],
    )


def kernel(phoneme_sequences, duration_indexes, output_length):
    table = phoneme_sequences.reshape(BATCH * X_STEPS, D_MODEL)
    len16 = jnp.pad(output_length.astype(jnp.int32), (0, LANES - BATCH))
    out = _sc_call()(table, duration_indexes, len16)
    return out.reshape(BATCH, Y_STEPS, D_MODEL)


# speedup vs baseline: 1.0114x; 1.0114x over previous
import functools

import jax
import jax.numpy as jnp
from jax import lax
from jax.experimental import pallas as pl
from jax.experimental.pallas import tpu as pltpu
from jax.experimental.pallas import tpu_sc as plsc

BATCH = 8
X_STEPS = 512
Y_STEPS = 4096
D_MODEL = 256

NC = 2
NS = 16
NW = NC * NS
LANES = 16

BLK = 128
BPB = Y_STEPS // BLK
NSLOT = BATCH
VPB = BLK // LANES
NBUF = 3
PRIME = NBUF - 1
ROT = NW // BATCH
ZROWS = 64


def _zero_rows(buf, lo, hi):
    zv = jnp.zeros((LANES,), jnp.float32)

    def body(r, carry):
        for c in range(D_MODEL // LANES):
            buf[r, pl.ds(c * LANES, LANES)] = zv
        return carry

    lax.fori_loop(lo, hi, body, 0)


def _sc_body(table_hbm, idx_hbm, len_hbm, out_hbm, *scratch):
    it = iter(scratch)
    idx_v = next(it)
    gidx = tuple(next(it) for _ in range(NSLOT))
    len_v = next(it)
    bufs = tuple(next(it) for _ in range(NBUF))
    zbuf = next(it)
    gsems = tuple(next(it) for _ in range(NBUF))
    wsems = tuple(next(it) for _ in range(NBUF))
    zsem = next(it)
    isem = next(it)

    w = lax.axis_index("s") * NC + lax.axis_index("c")

    pltpu.sync_copy(len_hbm, len_v)
    lenvec = len_v[...]

    len_s = [lenvec[k] for k in range(BATCH)]
    bat, pos, valid, rem, out_off = [], [], [], [], []
    for j in range(NSLOT):
        bj = jnp.mod(w + j, BATCH)
        lj = len_s[0]
        for k in range(1, BATCH):
            lj = jnp.where(bj == k, len_s[k], lj)
        p = jnp.mod(w - ROT * bj, BPB)
        r = jnp.clip(lj - p * BLK, 0, BLK)
        bat.append(bj)
        pos.append(p)
        rem.append(r)
        valid.append(r > 0)
        out_off.append(bj * Y_STEPS + p * BLK)
        pltpu.async_copy(idx_hbm.at[bj, pl.ds(p * BLK, BLK)],
                         idx_v.at[j], isem)

    def gather(j):
        pltpu.async_copy(table_hbm.at[gidx[j]], bufs[j % NBUF],
                         gsems[j % NBUF])

    def out_at(j):
        return out_hbm.at[pl.ds(out_off[j], BLK)]

    for j in range(NSLOT):
        pltpu.make_async_copy(idx_hbm.at[bat[j], pl.ds(pos[j] * BLK, BLK)],
                              idx_v.at[j], isem).wait()
        roff = lax.broadcast(bat[j] * X_STEPS, (LANES,))
        for v in range(VPB):
            g = idx_v[j, pl.ds(v * LANES, LANES)] + roff
            gidx[j][pl.ds(v * LANES, LANES)] = g
        if j < PRIME:
            @pl.when(valid[j])
            def _(j=j):
                gather(j)

    _zero_rows(zbuf, 0, ZROWS)

    for j in range(NSLOT):
        @pl.when(jnp.logical_not(valid[j]))
        def _(j=j):
            for h in range(BLK // ZROWS):
                pltpu.async_copy(
                    zbuf, out_hbm.at[pl.ds(out_off[j] + h * ZROWS, ZROWS)],
                    zsem)

    for j in range(NSLOT):
        @pl.when(valid[j])
        def _(j=j):
            pltpu.make_async_copy(table_hbm.at[gidx[j]], bufs[j % NBUF],
                                  gsems[j % NBUF]).wait()

        @pl.when(valid[j] & (rem[j] < BLK))
        def _(j=j):
            _zero_rows(bufs[j % NBUF], rem[j], BLK)

        @pl.when(valid[j])
        def _(j=j):
            pltpu.async_copy(bufs[j % NBUF], out_at(j), wsems[j % NBUF])

        if j + PRIME < NSLOT:
            k = j + PRIME - NBUF
            if k >= 0:
                @pl.when(valid[j + PRIME] & valid[k])
                def _(j=j, k=k):
                    pltpu.make_async_copy(bufs[k % NBUF], out_at(k),
                                          wsems[k % NBUF]).wait()

            @pl.when(valid[j + PRIME])
            def _(j=j):
                gather(j + PRIME)

    for k in range(NSLOT):
        cond = valid[k]
        if k + NBUF < NSLOT:
            cond = cond & jnp.logical_not(valid[k + NBUF])

        @pl.when(cond)
        def _(k=k):
            pltpu.make_async_copy(bufs[k % NBUF], out_at(k),
                                  wsems[k % NBUF]).wait()

    for j in range(NSLOT):
        @pl.when(jnp.logical_not(valid[j]))
        def _(j=j):
            for h in range(BLK // ZROWS):
                pltpu.make_async_copy(
                    zbuf, out_hbm.at[pl.ds(out_off[j] + h * ZROWS, ZROWS)],
                    zsem).wait()


@functools.cache
def _sc_call():
    mesh = plsc.VectorSubcoreMesh(
        core_axis_name="c", subcore_axis_name="s",
        num_cores=NC, num_subcores=NS)
    return pl.kernel(
        _sc_body,
        out_type=jax.ShapeDtypeStruct((BATCH * Y_STEPS, D_MODEL), jnp.float32),
        mesh=mesh,
        scratch_types=[
            pltpu.VMEM((NSLOT, BLK), jnp.int32),
            *[pltpu.VMEM((BLK,), jnp.int32) for _ in range(NSLOT)],
            pltpu.VMEM((LANES,), jnp.int32),
            *[pltpu.VMEM((BLK, D_MODEL), jnp.float32) for _ in range(NBUF)],
            pltpu.VMEM((ZROWS, D_MODEL), jnp.float32),
            *[pltpu.SemaphoreType.DMA for _ in range(2 * NBUF + 2)],
        ],
    )


def kernel(phoneme_sequences, duration_indexes, output_length):
    table = phoneme_sequences.reshape(BATCH * X_STEPS, D_MODEL)
    len16 = jnp.pad(output_length.astype(jnp.int32), (0, LANES - BATCH))
    out = _sc_call()(table, duration_indexes, len16)
    return out.reshape(BATCH, Y_STEPS, D_MODEL)
